# Initial kernel scaffold; baseline (speedup 1.0000x reference)
import jax
import jax.numpy as jnp
from jax.experimental import pallas as pl


def kernel(x, edge_index, W_rel1, b_rel1, W_root1, W_rel2, b_rel2, W_root2, W_gcn, b_gcn, W_lin, b_lin):
    raise NotImplementedError("write your pallas kernel here")


# speedup vs baseline: 6.0444x; 6.0444x over previous
import functools
import jax
import jax.numpy as jnp
from jax import lax
from jax.experimental import pallas as pl
from jax.experimental.pallas import tpu as pltpu
from jax.experimental.pallas import tpu_sc as plsc

NC = 2
NS = 16
NW = NC * NS
K = 128
D = 128


def _make_segsum(n_pad, e_pad, n_chunks):
    rows_per_tile = n_pad // NS
    edges_per_tile = e_pad // NW
    mesh = plsc.VectorSubcoreMesh(core_axis_name="c", subcore_axis_name="s")

    def body(table, src_h, dst_h, out_h, idx_v, didx_v, rows_v, acc, sem):
        c = lax.axis_index("c")
        s = lax.axis_index("s")
        z16 = jnp.zeros((16,), jnp.float32)

        def zero_row(r, _):
            for j in range(D // 16):
                rows_v[r, pl.ds(j * 16, 16)] = z16
            return 0
        lax.fori_loop(0, K, zero_row, 0)

        base_r = s * rows_per_tile
        off = 0
        while off < rows_per_tile:
            step = min(K, rows_per_tile - off)
            pltpu.sync_copy(rows_v.at[pl.ds(0, step)],
                            acc.at[pl.ds(base_r + off, step)])
            off += step
        plsc.subcore_barrier()

        wid = c * NS + s
        ebase = wid * edges_per_tile

        def chunk(i, _):
            o = ebase + i * K
            pltpu.sync_copy(src_h.at[pl.ds(o, K)], idx_v)
            pltpu.sync_copy(dst_h.at[pl.ds(o, K)], didx_v)
            pltpu.async_copy(table.at[idx_v], rows_v, sem).wait()
            pltpu.sync_copy(rows_v, acc.at[didx_v], add=True)
            return 0
        lax.fori_loop(0, n_chunks, chunk, 0)
        plsc.subcore_barrier()

        obase = c * n_pad + base_r
        pltpu.sync_copy(acc.at[pl.ds(base_r, rows_per_tile)],
                        out_h.at[pl.ds(obase, rows_per_tile)])

    return pl.kernel(
        body,
        out_type=[jax.ShapeDtypeStruct((NC * n_pad, D), jnp.float32)],
        mesh=mesh,
        scratch_types=[
            pltpu.VMEM((K,), jnp.int32),
            pltpu.VMEM((K,), jnp.int32),
            pltpu.VMEM((K, D), jnp.float32),
            pltpu.VMEM_SHARED((n_pad, D), jnp.float32),
            pltpu.SemaphoreType.DMA,
        ])


_BLK = 2000
_HB = 8000


def _histogram(lo3, hi3, nh):
    nb = lo3.shape[0]
    hb = lo3.shape[1]

    def body(lo_r, hi_r, o_r):
        i = pl.program_id(0)
        lo = lo_r[0]
        hi = hi_r[0]
        loh = (lax.broadcasted_iota(jnp.int32, (hb, 128), 1) == lo
               ).astype(jnp.float32)
        hih = (lax.broadcasted_iota(jnp.int32, (hb, nh), 1) == hi
               ).astype(jnp.float32)
        acc = lax.dot_general(hih, loh, (((0,), (0,)), ((), ())),
                              preferred_element_type=jnp.float32)

        @pl.when(i == 0)
        def _():
            o_r[...] = acc

        @pl.when(i > 0)
        def _():
            o_r[...] += acc

    blk = pl.BlockSpec((1, hb, 1), lambda i: (i, 0, 0))
    return pl.pallas_call(
        body,
        grid=(nb,),
        in_specs=[blk, blk],
        out_specs=pl.BlockSpec((nh, 128), lambda i: (0, 0)),
        out_shape=jax.ShapeDtypeStruct((nh, 128), jnp.float32),
    )(lo3, hi3)


def _graphconv_dense(p0, p1, x, wr_t, wt_t, b):
    n = x.shape[0]

    def body(p0_r, p1_r, x_r, wr, wt, b_r, o_r):
        a = p0_r[...] + p1_r[...]
        o = (jnp.dot(a, wr[...], preferred_element_type=jnp.float32)
             + jnp.dot(x_r[...], wt[...], preferred_element_type=jnp.float32)
             + b_r[...])
        o_r[...] = jnp.maximum(o, 0.0)

    row = pl.BlockSpec((_BLK, D), lambda i: (i, 0))
    full = pl.BlockSpec((D, D), lambda i: (0, 0))
    return pl.pallas_call(
        body,
        grid=(n // _BLK,),
        in_specs=[row, row, row, full, full,
                  pl.BlockSpec((1, D), lambda i: (0, 0))],
        out_specs=row,
        out_shape=jax.ShapeDtypeStruct((n, D), jnp.float32),
    )(p0, p1, x, wr_t, wt_t, b)


def _gcn_pre(cnt, h, wg_t):
    n = h.shape[0]

    def body(c_r, h_r, wg, y_r, dis_r):
        dis = lax.rsqrt(c_r[...] + 1.0)
        y_r[...] = dis * jnp.dot(h_r[...], wg[...],
                                 preferred_element_type=jnp.float32)
        dis_r[...] = jnp.broadcast_to(dis, (_BLK, 16))

    row = pl.BlockSpec((_BLK, D), lambda i: (i, 0))
    row16 = pl.BlockSpec((_BLK, 16), lambda i: (i, 0))
    return pl.pallas_call(
        body,
        grid=(n // _BLK,),
        in_specs=[pl.BlockSpec((_BLK, 1), lambda i: (i, 0)), row,
                  pl.BlockSpec((D, D), lambda i: (0, 0))],
        out_specs=[row, row16],
        out_shape=[jax.ShapeDtypeStruct((n, D), jnp.float32),
                   jax.ShapeDtypeStruct((n, 16), jnp.float32)],
    )(cnt, h, wg_t)


def _final_dense(q0, q1, y, dis16, b_gcn, wl_t, b_lin):
    n = y.shape[0]
    c = wl_t.shape[1]

    def body(q0_r, q1_r, y_r, dis_r, bg, wl, bl, o_r):
        g = dis_r[...][:, 0:1] * (q0_r[...] + q1_r[...] + y_r[...]) + bg[...]
        o_r[...] = jnp.dot(g, wl[...],
                           preferred_element_type=jnp.float32) + bl[...]

    row = pl.BlockSpec((_BLK, D), lambda i: (i, 0))
    row16 = pl.BlockSpec((_BLK, 16), lambda i: (i, 0))
    return pl.pallas_call(
        body,
        grid=(n // _BLK,),
        in_specs=[row, row, row, row16,
                  pl.BlockSpec((1, D), lambda i: (0, 0)),
                  pl.BlockSpec((D, c), lambda i: (0, 0)),
                  pl.BlockSpec((1, c), lambda i: (0, 0))],
        out_specs=pl.BlockSpec((_BLK, c), lambda i: (i, 0)),
        out_shape=jax.ShapeDtypeStruct((n, c), jnp.float32),
    )(q0, q1, y, dis16, b_gcn, wl_t, b_lin)


def kernel(x, edge_index, W_rel1, b_rel1, W_root1, W_rel2, b_rel2, W_root2,
           W_gcn, b_gcn, W_lin, b_lin):
    n, d = x.shape
    e = edge_index.shape[1]

    n_pad = (NS * 8) * (-(-n // (NS * 8)))
    n_chunks = -(-e // (NW * K))
    e_pad = NW * K * n_chunks
    pad = e_pad - e

    src = edge_index[0]
    dst = edge_index[1]
    if pad:
        src = jnp.concatenate([src, jnp.zeros((pad,), jnp.int32)])
        dst = jnp.concatenate([dst, jnp.full((pad,), n, jnp.int32)])

    nb = -(-e // _HB)
    pad_h = nb * _HB - e
    dst_h = edge_index[1]
    if pad_h:
        dst_h = jnp.concatenate([dst_h, jnp.full((pad_h,), n, jnp.int32)])
    nh = 8 * (-(-(n + 1) // (8 * 128)))
    counts = _histogram((dst_h % 128).reshape(nb, _HB, 1),
                        (dst_h // 128).reshape(nb, _HB, 1), nh)
    cnt = counts.reshape(-1)[:n].reshape(n, 1)

    segsum = _make_segsum(n_pad, e_pad, n_chunks)

    part, = segsum(x, src, dst)
    h1 = _graphconv_dense(part[:n], part[n_pad:n_pad + n], x,
                          W_rel1.T, W_root1.T, b_rel1.reshape(1, -1))

    part2, = segsum(h1, src, dst)
    h2 = _graphconv_dense(part2[:n], part2[n_pad:n_pad + n], h1,
                          W_rel2.T, W_root2.T, b_rel2.reshape(1, -1))

    y, dis16 = _gcn_pre(cnt, h2, W_gcn.T)
    part3, = segsum(y, src, dst)
    return _final_dense(part3[:n], part3[n_pad:n_pad + n], y, dis16,
                        b_gcn.reshape(1, -1), W_lin.T, b_lin.reshape(1, -1))
